# baseline (device time: 86224 ns/iter reference)
import jax
import jax.numpy as jnp
from jax import lax
from jax.experimental import pallas as pl
from jax.experimental.pallas import tpu as pltpu

N_DEV = 16


def kernel(x, W1, W2):
    m, k = x.shape
    h_per = W1.shape[1]
    n = W2.shape[1]
    chunk = m // N_DEV

    def body(x_ref, w1_ref, w2_ref, out_ref,
             acc_ref, sbuf_ref, rs_recv_ref,
             rs_send_sems, rs_recv_sems, ag_send_sems, ag_recv_sems):
        my = lax.axis_index("i")
        left = lax.rem(my + N_DEV - 1, N_DEV)
        right = lax.rem(my + 1, N_DEV)

        barrier_sem = pltpu.get_barrier_semaphore()
        pl.semaphore_signal(barrier_sem, inc=1, device_id=(left,),
                            device_id_type=pl.DeviceIdType.MESH)
        pl.semaphore_signal(barrier_sem, inc=1, device_id=(right,),
                            device_id_type=pl.DeviceIdType.MESH)
        pl.semaphore_wait(barrier_sem, 2)

        h = jnp.maximum(
            jnp.dot(x_ref[:, :], w1_ref[:, :],
                    preferred_element_type=jnp.float32),
            0.0,
        )
        acc_ref[:, :] = jnp.dot(h, w2_ref[:, :],
                                preferred_element_type=jnp.float32)

        for s in range(N_DEV - 1):
            c = lax.rem(my - s + N_DEV, N_DEV)
            if s == 0:
                sbuf_ref[:, :] = acc_ref[pl.ds(c * chunk, chunk), :]
            else:
                sbuf_ref[:, :] = (acc_ref[pl.ds(c * chunk, chunk), :]
                                  + rs_recv_ref[s - 1])
            rdma = pltpu.make_async_remote_copy(
                src_ref=sbuf_ref,
                dst_ref=rs_recv_ref.at[s],
                send_sem=rs_send_sems.at[s],
                recv_sem=rs_recv_sems.at[s],
                device_id=(right,),
                device_id_type=pl.DeviceIdType.MESH,
            )
            rdma.start()
            rdma.wait()

        c_fin = lax.rem(my + 1, N_DEV)
        out_ref[pl.ds(c_fin * chunk, chunk), :] = (
            acc_ref[pl.ds(c_fin * chunk, chunk), :] + rs_recv_ref[N_DEV - 2]
        )

        for s in range(N_DEV - 1):
            c = lax.rem(my + 1 - s + N_DEV, N_DEV)
            rdma = pltpu.make_async_remote_copy(
                src_ref=out_ref.at[pl.ds(c * chunk, chunk)],
                dst_ref=out_ref.at[pl.ds(c * chunk, chunk)],
                send_sem=ag_send_sems.at[s],
                recv_sem=ag_recv_sems.at[s],
                device_id=(right,),
                device_id_type=pl.DeviceIdType.MESH,
            )
            rdma.start()
            rdma.wait()

    return pl.pallas_call(
        body,
        out_shape=jax.ShapeDtypeStruct((m, n), jnp.float32),
        in_specs=[
            pl.BlockSpec(memory_space=pltpu.VMEM),
            pl.BlockSpec(memory_space=pltpu.VMEM),
            pl.BlockSpec(memory_space=pltpu.VMEM),
        ],
        out_specs=pl.BlockSpec(memory_space=pltpu.VMEM),
        scratch_shapes=[
            pltpu.VMEM((m, n), jnp.float32),
            pltpu.VMEM((chunk, n), jnp.float32),
            pltpu.VMEM((N_DEV - 1, chunk, n), jnp.float32),
            pltpu.SemaphoreType.DMA((N_DEV - 1,)),
            pltpu.SemaphoreType.DMA((N_DEV - 1,)),
            pltpu.SemaphoreType.DMA((N_DEV - 1,)),
            pltpu.SemaphoreType.DMA((N_DEV - 1,)),
        ],
        compiler_params=pltpu.CompilerParams(collective_id=0),
    )(x, W1, W2)


# device time: 45982 ns/iter; 1.8752x vs baseline; 1.8752x over previous
import jax
import jax.numpy as jnp
from jax import lax
from jax.experimental import pallas as pl
from jax.experimental.pallas import tpu as pltpu

N_DEV = 16


def _v_to_p(v):
    xp = v & 1
    yp = (v >> 1) & 1
    zp = v >> 2
    plane = (xp ^ yp) + 2 * yp
    return 4 * zp + plane


def kernel(x, W1, W2):
    m, k = x.shape
    n = W2.shape[1]

    def body(x_ref, w1_ref, w2_ref, out_ref,
             acc_ref, rs0_ref, rs1_ref, rs2_ref, rs3_ref,
             rs_send_sems, rs_recv_sems, ag_send_sems, ag_recv_sems):
        my = lax.axis_index("i")
        plane = lax.rem(my, 4)
        z = my // 4
        b0 = plane & 1
        b1 = (plane >> 1) & 1
        my_x = b0 ^ b1
        my_y = b1
        v = my_x + 2 * my_y + 4 * z

        partners = [_v_to_p(v ^ (1 << kk)) for kk in range(4)]

        barrier_sem = pltpu.get_barrier_semaphore()
        for p in partners:
            pl.semaphore_signal(barrier_sem, inc=1, device_id=(p,),
                                device_id_type=pl.DeviceIdType.MESH)
        pl.semaphore_wait(barrier_sem, 4)

        h = jnp.maximum(
            jnp.dot(x_ref[:, :], w1_ref[:, :],
                    preferred_element_type=jnp.float32),
            0.0,
        )
        acc_ref[:, :] = jnp.dot(h, w2_ref[:, :],
                                preferred_element_type=jnp.float32)

        rs_bufs = [rs0_ref, rs1_ref, rs2_ref, rs3_ref]

        s = my_x * 0
        for kk in range(4):
            half = 256 >> kk
            bit = (v >> kk) & 1
            send_start = s + (1 - bit) * half
            keep_start = s + bit * half
            rdma = pltpu.make_async_remote_copy(
                src_ref=acc_ref.at[pl.ds(send_start, half)],
                dst_ref=rs_bufs[kk],
                send_sem=rs_send_sems.at[kk],
                recv_sem=rs_recv_sems.at[kk],
                device_id=(partners[kk],),
                device_id_type=pl.DeviceIdType.MESH,
            )
            rdma.start()
            rdma.wait()
            acc_ref[pl.ds(keep_start, half), :] = (
                acc_ref[pl.ds(keep_start, half), :] + rs_bufs[kk][:, :]
            )
            s = keep_start

        out_ref[pl.ds(s, 32), :] = acc_ref[pl.ds(s, 32), :]

        cur = s
        for kk in (3, 2, 1, 0):
            size = 256 >> kk
            bit = (v >> kk) & 1
            rdma = pltpu.make_async_remote_copy(
                src_ref=out_ref.at[pl.ds(cur, size)],
                dst_ref=out_ref.at[pl.ds(cur, size)],
                send_sem=ag_send_sems.at[kk],
                recv_sem=ag_recv_sems.at[kk],
                device_id=(partners[kk],),
                device_id_type=pl.DeviceIdType.MESH,
            )
            rdma.start()
            rdma.wait()
            cur = cur - bit * size

    return pl.pallas_call(
        body,
        out_shape=jax.ShapeDtypeStruct((m, n), jnp.float32),
        in_specs=[
            pl.BlockSpec(memory_space=pltpu.VMEM),
            pl.BlockSpec(memory_space=pltpu.VMEM),
            pl.BlockSpec(memory_space=pltpu.VMEM),
        ],
        out_specs=pl.BlockSpec(memory_space=pltpu.VMEM),
        scratch_shapes=[
            pltpu.VMEM((m, n), jnp.float32),
            pltpu.VMEM((256, n), jnp.float32),
            pltpu.VMEM((128, n), jnp.float32),
            pltpu.VMEM((64, n), jnp.float32),
            pltpu.VMEM((32, n), jnp.float32),
            pltpu.SemaphoreType.DMA((4,)),
            pltpu.SemaphoreType.DMA((4,)),
            pltpu.SemaphoreType.DMA((4,)),
            pltpu.SemaphoreType.DMA((4,)),
        ],
        compiler_params=pltpu.CompilerParams(collective_id=0),
    )(x, W1, W2)


# device time: 42594 ns/iter; 2.0243x vs baseline; 1.0795x over previous
import jax
import jax.numpy as jnp
from jax import lax
from jax.experimental import pallas as pl
from jax.experimental.pallas import tpu as pltpu

N_DEV = 16


def _v_to_p(v):
    xp = v & 1
    yp = (v >> 1) & 1
    zp = v >> 2
    plane = (xp ^ yp) + 2 * yp
    return 4 * zp + plane


def kernel(x, W1, W2):
    m, k = x.shape
    n = W2.shape[1]

    def body(x_ref, w1_ref, w2_ref, out_ref,
             acc_ref, rs0_ref, rs1_ref, rs2_ref, rs3_ref,
             rs_send_sems, rs_recv_sems, ag_send_sems, ag_recv_sems):
        my = lax.axis_index("i")
        plane = lax.rem(my, 4)
        z = my // 4
        b0 = plane & 1
        b1 = (plane >> 1) & 1
        my_x = b0 ^ b1
        my_y = b1
        v = my_x + 2 * my_y + 4 * z

        partners = [_v_to_p(v ^ (1 << kk)) for kk in range(4)]

        barrier_sem = pltpu.get_barrier_semaphore()
        for p in partners:
            pl.semaphore_signal(barrier_sem, inc=1, device_id=(p,),
                                device_id_type=pl.DeviceIdType.MESH)
        pl.semaphore_wait(barrier_sem, 4)

        def compute_rows(start, nrows):
            h = jnp.maximum(
                jnp.dot(x_ref[pl.ds(start, nrows), :], w1_ref[:, :],
                        preferred_element_type=jnp.float32),
                0.0,
            )
            acc_ref[pl.ds(start, nrows), :] = jnp.dot(
                h, w2_ref[:, :], preferred_element_type=jnp.float32)

        rs_bufs = [rs0_ref, rs1_ref, rs2_ref, rs3_ref]
        pending = []

        bit0 = v & 1
        send_start0 = (1 - bit0) * 256
        keep_start0 = bit0 * 256
        compute_rows(send_start0, 256)
        rdma0 = pltpu.make_async_remote_copy(
            src_ref=acc_ref.at[pl.ds(send_start0, 256)],
            dst_ref=rs0_ref,
            send_sem=rs_send_sems.at[0],
            recv_sem=rs_recv_sems.at[0],
            device_id=(partners[0],),
            device_id_type=pl.DeviceIdType.MESH,
        )
        rdma0.start()
        pending.append(rdma0)
        compute_rows(keep_start0, 256)
        rdma0.wait_recv()
        acc_ref[pl.ds(keep_start0, 256), :] = (
            acc_ref[pl.ds(keep_start0, 256), :] + rs0_ref[:, :]
        )
        s = keep_start0

        for kk in range(1, 4):
            half = 256 >> kk
            bit = (v >> kk) & 1
            send_start = s + (1 - bit) * half
            keep_start = s + bit * half
            rdma = pltpu.make_async_remote_copy(
                src_ref=acc_ref.at[pl.ds(send_start, half)],
                dst_ref=rs_bufs[kk],
                send_sem=rs_send_sems.at[kk],
                recv_sem=rs_recv_sems.at[kk],
                device_id=(partners[kk],),
                device_id_type=pl.DeviceIdType.MESH,
            )
            rdma.start()
            pending.append(rdma)
            rdma.wait_recv()
            acc_ref[pl.ds(keep_start, half), :] = (
                acc_ref[pl.ds(keep_start, half), :] + rs_bufs[kk][:, :]
            )
            s = keep_start

        out_ref[pl.ds(s, 32), :] = acc_ref[pl.ds(s, 32), :]

        cur = s
        for kk in (3, 2, 1, 0):
            size = 256 >> kk
            bit = (v >> kk) & 1
            rdma = pltpu.make_async_remote_copy(
                src_ref=out_ref.at[pl.ds(cur, size)],
                dst_ref=out_ref.at[pl.ds(cur, size)],
                send_sem=ag_send_sems.at[kk],
                recv_sem=ag_recv_sems.at[kk],
                device_id=(partners[kk],),
                device_id_type=pl.DeviceIdType.MESH,
            )
            rdma.start()
            pending.append(rdma)
            rdma.wait_recv()
            cur = cur - bit * size

        for rdma in pending:
            rdma.wait_send()

    return pl.pallas_call(
        body,
        out_shape=jax.ShapeDtypeStruct((m, n), jnp.float32),
        in_specs=[
            pl.BlockSpec(memory_space=pltpu.VMEM),
            pl.BlockSpec(memory_space=pltpu.VMEM),
            pl.BlockSpec(memory_space=pltpu.VMEM),
        ],
        out_specs=pl.BlockSpec(memory_space=pltpu.VMEM),
        scratch_shapes=[
            pltpu.VMEM((m, n), jnp.float32),
            pltpu.VMEM((256, n), jnp.float32),
            pltpu.VMEM((128, n), jnp.float32),
            pltpu.VMEM((64, n), jnp.float32),
            pltpu.VMEM((32, n), jnp.float32),
            pltpu.SemaphoreType.DMA((4,)),
            pltpu.SemaphoreType.DMA((4,)),
            pltpu.SemaphoreType.DMA((4,)),
            pltpu.SemaphoreType.DMA((4,)),
        ],
        compiler_params=pltpu.CompilerParams(collective_id=0),
    )(x, W1, W2)
